# baseline (speedup 1.0000x reference)
import functools

import jax
import jax.numpy as jnp
from jax import lax
from jax.experimental import pallas as pl
from jax.experimental.pallas import tpu as pltpu
from jax.experimental.pallas import tpu_sc as plsc

D = 1024
H = 16
DH = 64
NHASH = 4
NBKT = 32
HB = 16
BS = 64
EPS = 1e-5
_OW = DH + 64


def _qkv_body(x_ref, g_ref, b_ref, wqk_ref, wv_ref, qk_ref, v_ref):
    x = x_ref[...]
    mu = jnp.mean(x, -1, keepdims=True)
    var = jnp.mean((x - mu) ** 2, -1, keepdims=True)
    h = (x - mu) / jnp.sqrt(var + EPS) * g_ref[...] + b_ref[...]
    qk_ref[...] = jnp.dot(h, wqk_ref[...], preferred_element_type=jnp.float32)
    v_ref[...] = jnp.dot(h, wv_ref[...], preferred_element_type=jnp.float32)


def _qkv_proj(x2, g, b, Wqk, Wv, S, blk=256):
    grid = (S // blk,)
    return pl.pallas_call(
        _qkv_body,
        grid=grid,
        in_specs=[
            pl.BlockSpec((blk, D), lambda i: (i, 0)),
            pl.BlockSpec((1, D), lambda i: (0, 0)),
            pl.BlockSpec((1, D), lambda i: (0, 0)),
            pl.BlockSpec((D, D), lambda i: (0, 0)),
            pl.BlockSpec((D, D), lambda i: (0, 0)),
        ],
        out_specs=[
            pl.BlockSpec((blk, D), lambda i: (i, 0)),
            pl.BlockSpec((blk, D), lambda i: (i, 0)),
        ],
        out_shape=[
            jax.ShapeDtypeStruct((S, D), jnp.float32),
            jax.ShapeDtypeStruct((S, D), jnp.float32),
        ],
    )(x2, g.reshape(1, D), b.reshape(1, D), Wqk, Wv)


def _bucket_body(qk_ref, rot_ref, bid_ref):
    S = qk_ref.shape[1]
    r = jnp.dot(qk_ref[0], rot_ref[...], preferred_element_type=jnp.float32)
    cols = []
    for h in range(NHASH):
        seg = r[:, h * 2 * HB:(h + 1) * 2 * HB]
        b = jnp.argmax(seg, axis=-1, keepdims=True).astype(jnp.int32)
        cols.append(b + h * NBKT)
    bid_ref[0] = jnp.concatenate(cols, axis=-1)


def _bucket_ids(qk_heads, rotf, S):
    return pl.pallas_call(
        _bucket_body,
        grid=(H,),
        in_specs=[
            pl.BlockSpec((1, S, DH), lambda h: (h, 0, 0)),
            pl.BlockSpec((DH, NHASH * 2 * HB), lambda h: (0, 0)),
        ],
        out_specs=pl.BlockSpec((1, S, NHASH), lambda h: (h, 0, 0)),
        out_shape=jax.ShapeDtypeStruct((H, S, NHASH), jnp.int32),
    )(qk_heads, rotf)


_CSR = 256


def _count_body(b_ref, dest_ref, *, n, nk):
    g = pl.program_id(0)
    b_col = b_ref[0]
    iota_k = jax.lax.broadcasted_iota(jnp.int32, (1, nk), 1)
    O = (b_col == iota_k).astype(jnp.float32)
    nb = n // _CSR
    O3 = O.reshape(nb, _CSR, nk)
    Hg = jnp.sum(O3, axis=1)
    gi = jax.lax.broadcasted_iota(jnp.int32, (nb, nb), 0)
    gj = jax.lax.broadcasted_iota(jnp.int32, (nb, nb), 1)
    Tg = (gj < gi).astype(jnp.float32)
    Hcum = jnp.dot(Tg, Hg, preferred_element_type=jnp.float32)
    total = jnp.sum(Hg, axis=0, keepdims=True)
    ki = jax.lax.broadcasted_iota(jnp.int32, (nk, nk), 0)
    kj = jax.lax.broadcasted_iota(jnp.int32, (nk, nk), 1)
    Mu = (ki < kj).astype(jnp.float32)
    startk = jnp.dot(total, Mu, preferred_element_type=jnp.float32)
    offs = Hcum + startk
    ri = jax.lax.broadcasted_iota(jnp.int32, (_CSR, _CSR), 0)
    rj = jax.lax.broadcasted_iota(jnp.int32, (_CSR, _CSR), 1)
    L = (rj < ri).astype(jnp.float32)
    for blk in range(nb):
        Og = O3[blk]
        P = jnp.dot(L, Og, preferred_element_type=jnp.float32)
        dg = jnp.sum(Og * (P + offs[blk:blk + 1, :]), axis=1, keepdims=True)
        dest_ref[0, pl.ds(blk * _CSR, _CSR)] = dg.astype(jnp.int32)


def _count_sort(b_flat):
    n = b_flat.shape[1]
    nk = NHASH * NBKT
    body = functools.partial(_count_body, n=n, nk=nk)
    out = pl.pallas_call(
        body,
        grid=(H,),
        in_specs=[pl.BlockSpec((1, n, 1), lambda h: (h, 0, 0))],
        out_specs=pl.BlockSpec((1, n, 1), lambda h: (h, 0, 0)),
        out_shape=jax.ShapeDtypeStruct((H, n, 1), jnp.int32),
    )(b_flat.reshape(H, n, 1))
    return out.reshape(H, n)


def _attn_body(sqkv_ref, stc_ref, out_ref, *, nc, cb):
    ncb = nc // cb

    def prep(blk):
        qk = blk[..., :DH]
        v = blk[..., DH:]
        nrm = jnp.sqrt(jnp.sum(qk * qk, -1, keepdims=True)) + 1e-9
        return qk / nrm, v

    def body(i, _):
        c0 = i * cb
        blk = sqkv_ref[0, pl.ds(c0, cb)]
        q = blk[..., :DH]
        prev_c = jnp.where(c0 == 0, nc - 1, c0 - 1)
        pblk = sqkv_ref[0, pl.ds(prev_c, 1)]
        k_c, v_c = prep(blk)
        k_p1, v_p1 = prep(pblk)
        k_prev = jnp.concatenate([k_p1, k_c[:-1]], 0)
        v_prev = jnp.concatenate([v_p1, v_c[:-1]], 0)
        t_c = stc_ref[0, pl.ds(c0, cb)]
        t_p1 = stc_ref[0, pl.ds(prev_c, 1)]
        t_prev = jnp.concatenate([t_p1, t_c[:-1]], 0)
        bk = jnp.concatenate([k_c, k_prev], 1)
        bv = jnp.concatenate([v_c, v_prev], 1)
        bt = jnp.concatenate([t_c, t_prev], 1)
        dots = jax.lax.dot_general(
            q, bk, (((2,), (2,)), ((0,), (0,))),
            preferred_element_type=jnp.float32) * (DH ** -0.5)
        mask = t_c[:, :, None] == bt[:, None, :]
        dots = jnp.where(mask, -1e5, dots)
        m = jnp.max(dots, -1, keepdims=True)
        lse = m + jnp.log(jnp.sum(jnp.exp(dots - m), -1, keepdims=True))
        p = jnp.exp(dots - lse)
        o = jax.lax.dot_general(
            p, bv, (((2,), (1,)), ((0,), (0,))),
            preferred_element_type=jnp.float32)
        out_ref[0, pl.ds(c0, cb), :, :DH] = o
        out_ref[0, pl.ds(c0, cb), :, DH:DH + 1] = lse
        return 0

    jax.lax.fori_loop(0, ncb, body, 0)


def _attention(sqkv, stc, nc, cb=16):
    body = functools.partial(_attn_body, nc=nc, cb=cb)
    return pl.pallas_call(
        body,
        grid=(H,),
        in_specs=[
            pl.BlockSpec((1, nc, BS, 2 * DH), lambda h: (h, 0, 0, 0)),
            pl.BlockSpec((1, nc, BS), lambda h: (h, 0, 0)),
        ],
        out_specs=pl.BlockSpec((1, nc, BS, _OW), lambda h: (h, 0, 0, 0)),
        out_shape=jax.ShapeDtypeStruct((H, nc, BS, _OW), jnp.float32),
    )(sqkv, stc)


def _comb_body(u_ref, wo_ref, x1_ref, bo_ref, out_ref):
    h = pl.program_id(0)
    u = u_ref[0]
    so = u[..., :DH]
    sl = u[..., DH:DH + 1]
    m = jnp.max(sl, 0, keepdims=True)
    lse = m + jnp.log(jnp.sum(jnp.exp(sl - m), 0, keepdims=True))
    p = jnp.exp(sl - lse)
    o = jnp.sum(so * p, 0)
    acc = jnp.dot(o, wo_ref[0], preferred_element_type=jnp.float32)

    @pl.when(h == 0)
    def _():
        out_ref[...] = x1_ref[...] + bo_ref[...] + acc

    @pl.when(h != 0)
    def _():
        out_ref[...] += acc


def _combine_wo(u, Wo, x1, bo, S):
    return pl.pallas_call(
        _comb_body,
        grid=(H,),
        in_specs=[
            pl.BlockSpec((1, NHASH, S, _OW), lambda h: (h, 0, 0, 0)),
            pl.BlockSpec((1, DH, D), lambda h: (h, 0, 0)),
            pl.BlockSpec((S, D), lambda h: (0, 0)),
            pl.BlockSpec((1, D), lambda h: (0, 0)),
        ],
        out_specs=pl.BlockSpec((S, D), lambda h: (0, 0)),
        out_shape=jax.ShapeDtypeStruct((S, D), jnp.float32),
        compiler_params=pltpu.CompilerParams(
            dimension_semantics=("arbitrary",)),
    )(u, Wo.reshape(H, DH, D), x1, bo.reshape(1, D))


def _ffn_body(y1_ref, x2_ref, g_ref, b_ref, w1_ref, b1_ref, w2_ref, b2_ref,
              out_ref, *, add_y1):
    kb = pl.program_id(1)
    x = y1_ref[...]
    mu = jnp.mean(x, -1, keepdims=True)
    var = jnp.mean((x - mu) ** 2, -1, keepdims=True)
    hh = (x - mu) / jnp.sqrt(var + EPS) * g_ref[...] + b_ref[...]
    a = jnp.dot(hh, w1_ref[...], preferred_element_type=jnp.float32) + b1_ref[...]
    ge = 0.5 * a * (1.0 + jax.lax.erf(a * (2.0 ** -0.5)))
    part = jnp.dot(ge, w2_ref[...], preferred_element_type=jnp.float32)

    @pl.when(kb == 0)
    def _():
        base = x2_ref[...] + b2_ref[...]
        if add_y1:
            base = base + x
        out_ref[...] = base + part

    @pl.when(kb != 0)
    def _():
        out_ref[...] += part


def _ffn(y1, x2, g, b, W1, b1, W2, b2, S, add_y1, blk=256, kb=1024):
    DF = W1.shape[-1]
    body = functools.partial(_ffn_body, add_y1=add_y1)
    return pl.pallas_call(
        body,
        grid=(S // blk, DF // kb),
        in_specs=[
            pl.BlockSpec((blk, D), lambda i, j: (i, 0)),
            pl.BlockSpec((blk, D), lambda i, j: (i, 0)),
            pl.BlockSpec((1, D), lambda i, j: (0, 0)),
            pl.BlockSpec((1, D), lambda i, j: (0, 0)),
            pl.BlockSpec((D, kb), lambda i, j: (0, j)),
            pl.BlockSpec((1, kb), lambda i, j: (0, j)),
            pl.BlockSpec((kb, D), lambda i, j: (j, 0)),
            pl.BlockSpec((1, D), lambda i, j: (0, 0)),
        ],
        out_specs=pl.BlockSpec((blk, D), lambda i, j: (i, 0)),
        out_shape=jax.ShapeDtypeStruct((S, D), jnp.float32),
        compiler_params=pltpu.CompilerParams(
            dimension_semantics=("arbitrary", "arbitrary")),
    )(y1, x2, g.reshape(1, D), b.reshape(1, D), W1, b1.reshape(1, DF), W2,
      b2.reshape(1, D))


_NW = 32
_CHUNK = 512


def _sc_gather(table, idx, W, dtype=jnp.float32):
    N = idx.shape[0]
    n_per = N // _NW
    nch = n_per // _CHUNK
    mesh = plsc.VectorSubcoreMesh(core_axis_name="c", subcore_axis_name="s")

    @functools.partial(
        pl.kernel, mesh=mesh,
        out_type=jax.ShapeDtypeStruct((N, W), dtype),
        scratch_types=[
            pltpu.VMEM((_CHUNK,), jnp.int32),
            pltpu.VMEM((_CHUNK, W), dtype),
            pltpu.SemaphoreType.DMA,
        ],
    )
    def k(table_hbm, idx_hbm, out_hbm, idx_v, rows_v, sem):
        wid = lax.axis_index("s") * 2 + lax.axis_index("c")
        base = wid * n_per
        for c in range(nch):
            off = base + c * _CHUNK
            pltpu.sync_copy(idx_hbm.at[pl.ds(off, _CHUNK)], idx_v)
            pltpu.async_copy(table_hbm.at[idx_v], rows_v, sem).wait()
            pltpu.sync_copy(rows_v, out_hbm.at[pl.ds(off, _CHUNK)])

    return k(table, idx)


def _sc_scatter(rows, idx, W, dtype=jnp.float32):
    N = idx.shape[0]
    n_per = N // _NW
    nch = n_per // _CHUNK
    mesh = plsc.VectorSubcoreMesh(core_axis_name="c", subcore_axis_name="s")

    @functools.partial(
        pl.kernel, mesh=mesh,
        out_type=jax.ShapeDtypeStruct((N, W), dtype),
        scratch_types=[
            pltpu.VMEM((_CHUNK,), jnp.int32),
            pltpu.VMEM((_CHUNK, W), dtype),
            pltpu.SemaphoreType.DMA,
        ],
    )
    def k(rows_hbm, idx_hbm, out_hbm, idx_v, rows_v, sem):
        wid = lax.axis_index("s") * 2 + lax.axis_index("c")
        base = wid * n_per
        for c in range(nch):
            off = base + c * _CHUNK
            pltpu.sync_copy(idx_hbm.at[pl.ds(off, _CHUNK)], idx_v)
            pltpu.sync_copy(rows_hbm.at[pl.ds(off, _CHUNK)], rows_v)
            pltpu.async_copy(rows_v, out_hbm.at[idx_v], sem).wait()

    return k(rows, idx)


def _gather_rows(qkv, st, S):
    gidx = (st + (jnp.arange(H, dtype=jnp.int32) * S)[:, None]).reshape(-1)
    out = _sc_gather(qkv.reshape(H * S, 2 * DH), gidx, 2 * DH)
    return out.reshape(H, st.shape[1], 2 * DH)


def _attn_layer(x1, x2, g, b, Wqk, Wv, Wo, bo, rot, S):
    nc = NHASH * NBKT
    n = NHASH * S
    qk2d, v2d = _qkv_proj(x2, g, b, Wqk, Wv, S)
    qkh = qk2d.reshape(S, H, DH).transpose(1, 0, 2)
    vh = v2d.reshape(S, H, DH).transpose(1, 0, 2)
    rotf = jnp.concatenate([rot, -rot], axis=-1).reshape(DH, NHASH * 2 * HB)
    bid = _bucket_ids(qkh, rotf, S)
    b_flat = bid.transpose(0, 2, 1).reshape(H, n)
    dest = _count_sort(b_flat)
    didx = (dest + (jnp.arange(H, dtype=jnp.int32) * n)[:, None]).reshape(-1)
    pos = (jnp.arange(n, dtype=jnp.int32) % S)
    st = jax.vmap(
        lambda d: jnp.zeros((n,), jnp.int32).at[d].set(pos))(dest)
    qkv = jnp.concatenate([qkh, vh], axis=-1)
    sqkv = _gather_rows(qkv, st, S).reshape(H, nc, BS, 2 * DH)
    stc = st.reshape(H, nc, BS)
    attn = _attention(sqkv, stc, nc)
    u = _sc_gather(attn.reshape(H * n, _OW), didx, _OW)
    u = u.reshape(H, NHASH, S, _OW)
    return _combine_wo(u, Wo, x1, bo, S)


def kernel(x, gf, bf, Wqk, Wv, Wo, bo, gg, bg, W1, b1, W2, b2, rot):
    S = x.shape[1]
    x1 = x[0]
    x2 = x[0]
    depth = gf.shape[0]
    for l in range(depth):
        y1 = _attn_layer(x1, x2, gf[l], bf[l], Wqk[l], Wv[l], Wo[l], bo[l],
                         rot[l], S)
        y2 = _ffn(y1, x2, gg[l], bg[l], W1[l], b1[l], W2[l], b2[l], S,
                  add_y1=(l == depth - 1))
        x1, x2 = y1, y2
    return x2[None]


# speedup vs baseline: 1.9329x; 1.9329x over previous
import functools

import jax
import jax.numpy as jnp
from jax import lax
from jax.experimental import pallas as pl
from jax.experimental.pallas import tpu as pltpu
from jax.experimental.pallas import tpu_sc as plsc

D = 1024
H = 16
DH = 64
NHASH = 4
NBKT = 32
HB = 16
BS = 64
EPS = 1e-5
_OW = DH + 64


def _qkv_body(x_ref, g_ref, b_ref, wqk_ref, wv_ref, qk_ref, v_ref):
    x = x_ref[...]
    mu = jnp.mean(x, -1, keepdims=True)
    var = jnp.mean((x - mu) ** 2, -1, keepdims=True)
    h = (x - mu) / jnp.sqrt(var + EPS) * g_ref[...] + b_ref[...]
    qk_ref[...] = jnp.dot(h, wqk_ref[...], preferred_element_type=jnp.float32)
    v_ref[...] = jnp.dot(h, wv_ref[...], preferred_element_type=jnp.float32)


def _qkv_proj(x2, g, b, Wqk, Wv, S, blk=256):
    grid = (S // blk,)
    return pl.pallas_call(
        _qkv_body,
        grid=grid,
        in_specs=[
            pl.BlockSpec((blk, D), lambda i: (i, 0)),
            pl.BlockSpec((1, D), lambda i: (0, 0)),
            pl.BlockSpec((1, D), lambda i: (0, 0)),
            pl.BlockSpec((D, D), lambda i: (0, 0)),
            pl.BlockSpec((D, D), lambda i: (0, 0)),
        ],
        out_specs=[
            pl.BlockSpec((blk, D), lambda i: (i, 0)),
            pl.BlockSpec((blk, D), lambda i: (i, 0)),
        ],
        out_shape=[
            jax.ShapeDtypeStruct((S, D), jnp.float32),
            jax.ShapeDtypeStruct((S, D), jnp.float32),
        ],
    )(x2, g.reshape(1, D), b.reshape(1, D), Wqk, Wv)


def _bucket_body(qk_ref, rot_ref, bid_ref):
    S = qk_ref.shape[1]
    r = jnp.dot(qk_ref[0], rot_ref[...], preferred_element_type=jnp.float32)
    cols = []
    for h in range(NHASH):
        seg = r[:, h * 2 * HB:(h + 1) * 2 * HB]
        b = jnp.argmax(seg, axis=-1, keepdims=True).astype(jnp.int32)
        cols.append(b + h * NBKT)
    bid_ref[0] = jnp.concatenate(cols, axis=-1)


def _bucket_ids(qk_heads, rotf, S):
    return pl.pallas_call(
        _bucket_body,
        grid=(H,),
        in_specs=[
            pl.BlockSpec((1, S, DH), lambda h: (h, 0, 0)),
            pl.BlockSpec((DH, NHASH * 2 * HB), lambda h: (0, 0)),
        ],
        out_specs=pl.BlockSpec((1, S, NHASH), lambda h: (h, 0, 0)),
        out_shape=jax.ShapeDtypeStruct((H, S, NHASH), jnp.int32),
    )(qk_heads, rotf)


_CSR = 256


def _count_body(b_ref, dest_ref, *, n, nk):
    g = pl.program_id(0)
    b_col = b_ref[0]
    iota_k = jax.lax.broadcasted_iota(jnp.int32, (1, nk), 1)
    O = (b_col == iota_k).astype(jnp.float32)
    nb = n // _CSR
    O3 = O.reshape(nb, _CSR, nk)
    Hg = jnp.sum(O3, axis=1)
    gi = jax.lax.broadcasted_iota(jnp.int32, (nb, nb), 0)
    gj = jax.lax.broadcasted_iota(jnp.int32, (nb, nb), 1)
    Tg = (gj < gi).astype(jnp.float32)
    Hcum = jnp.dot(Tg, Hg, preferred_element_type=jnp.float32)
    total = jnp.sum(Hg, axis=0, keepdims=True)
    ki = jax.lax.broadcasted_iota(jnp.int32, (nk, nk), 0)
    kj = jax.lax.broadcasted_iota(jnp.int32, (nk, nk), 1)
    Mu = (ki < kj).astype(jnp.float32)
    startk = jnp.dot(total, Mu, preferred_element_type=jnp.float32)
    offs = Hcum + startk
    ri = jax.lax.broadcasted_iota(jnp.int32, (_CSR, _CSR), 0)
    rj = jax.lax.broadcasted_iota(jnp.int32, (_CSR, _CSR), 1)
    L = (rj < ri).astype(jnp.float32)
    for blk in range(nb):
        Og = O3[blk]
        P = jnp.dot(L, Og, preferred_element_type=jnp.float32)
        dg = jnp.sum(Og * (P + offs[blk:blk + 1, :]), axis=1, keepdims=True)
        dest_ref[0, pl.ds(blk * _CSR, _CSR)] = dg.astype(jnp.int32)


def _count_sort(b_flat):
    n = b_flat.shape[1]
    nk = NHASH * NBKT
    body = functools.partial(_count_body, n=n, nk=nk)
    out = pl.pallas_call(
        body,
        grid=(H,),
        in_specs=[pl.BlockSpec((1, n, 1), lambda h: (h, 0, 0))],
        out_specs=pl.BlockSpec((1, n, 1), lambda h: (h, 0, 0)),
        out_shape=jax.ShapeDtypeStruct((H, n, 1), jnp.int32),
    )(b_flat.reshape(H, n, 1))
    return out.reshape(H, n)


def _attn_body(sqkv_ref, stc_ref, out_ref, *, nc, cb):
    ncb = nc // cb

    def prep(blk):
        qk = blk[..., :DH]
        v = blk[..., DH:]
        nrm = jnp.sqrt(jnp.sum(qk * qk, -1, keepdims=True)) + 1e-9
        return qk / nrm, v

    def body(i, _):
        c0 = i * cb
        blk = sqkv_ref[0, pl.ds(c0, cb)]
        q = blk[..., :DH]
        prev_c = jnp.where(c0 == 0, nc - 1, c0 - 1)
        pblk = sqkv_ref[0, pl.ds(prev_c, 1)]
        k_c, v_c = prep(blk)
        k_p1, v_p1 = prep(pblk)
        k_prev = jnp.concatenate([k_p1, k_c[:-1]], 0)
        v_prev = jnp.concatenate([v_p1, v_c[:-1]], 0)
        t_c = stc_ref[0, pl.ds(c0, cb)]
        t_p1 = stc_ref[0, pl.ds(prev_c, 1)]
        t_prev = jnp.concatenate([t_p1, t_c[:-1]], 0)
        bk = jnp.concatenate([k_c, k_prev], 1)
        bv = jnp.concatenate([v_c, v_prev], 1)
        bt = jnp.concatenate([t_c, t_prev], 1)
        dots = jax.lax.dot_general(
            q, bk, (((2,), (2,)), ((0,), (0,))),
            preferred_element_type=jnp.float32) * (DH ** -0.5)
        mask = t_c[:, :, None] == bt[:, None, :]
        dots = jnp.where(mask, -1e5, dots)
        m = jnp.max(dots, -1, keepdims=True)
        lse = m + jnp.log(jnp.sum(jnp.exp(dots - m), -1, keepdims=True))
        p = jnp.exp(dots - lse)
        o = jax.lax.dot_general(
            p, bv, (((2,), (1,)), ((0,), (0,))),
            preferred_element_type=jnp.float32)
        out_ref[0, pl.ds(c0, cb), :, :DH] = o
        out_ref[0, pl.ds(c0, cb), :, DH:DH + 1] = lse
        return 0

    jax.lax.fori_loop(0, ncb, body, 0)


def _attention(sqkv, stc, nc, cb=16):
    body = functools.partial(_attn_body, nc=nc, cb=cb)
    return pl.pallas_call(
        body,
        grid=(H,),
        in_specs=[
            pl.BlockSpec((1, nc, BS, 2 * DH), lambda h: (h, 0, 0, 0)),
            pl.BlockSpec((1, nc, BS), lambda h: (h, 0, 0)),
        ],
        out_specs=pl.BlockSpec((1, nc, BS, _OW), lambda h: (h, 0, 0, 0)),
        out_shape=jax.ShapeDtypeStruct((H, nc, BS, _OW), jnp.float32),
    )(sqkv, stc)


def _comb_body(u_ref, wo_ref, x1_ref, bo_ref, out_ref):
    h = pl.program_id(0)
    u = u_ref[0]
    so = u[..., :DH]
    sl = u[..., DH:DH + 1]
    m = jnp.max(sl, 0, keepdims=True)
    lse = m + jnp.log(jnp.sum(jnp.exp(sl - m), 0, keepdims=True))
    p = jnp.exp(sl - lse)
    o = jnp.sum(so * p, 0)
    acc = jnp.dot(o, wo_ref[0], preferred_element_type=jnp.float32)

    @pl.when(h == 0)
    def _():
        out_ref[...] = x1_ref[...] + bo_ref[...] + acc

    @pl.when(h != 0)
    def _():
        out_ref[...] += acc


def _combine_wo(u, Wo, x1, bo, S):
    return pl.pallas_call(
        _comb_body,
        grid=(H,),
        in_specs=[
            pl.BlockSpec((1, NHASH, S, _OW), lambda h: (h, 0, 0, 0)),
            pl.BlockSpec((1, DH, D), lambda h: (h, 0, 0)),
            pl.BlockSpec((S, D), lambda h: (0, 0)),
            pl.BlockSpec((1, D), lambda h: (0, 0)),
        ],
        out_specs=pl.BlockSpec((S, D), lambda h: (0, 0)),
        out_shape=jax.ShapeDtypeStruct((S, D), jnp.float32),
        compiler_params=pltpu.CompilerParams(
            dimension_semantics=("arbitrary",)),
    )(u, Wo.reshape(H, DH, D), x1, bo.reshape(1, D))


def _ffn_body(y1_ref, x2_ref, g_ref, b_ref, w1_ref, b1_ref, w2_ref, b2_ref,
              out_ref, *, add_y1):
    kb = pl.program_id(1)
    x = y1_ref[...]
    mu = jnp.mean(x, -1, keepdims=True)
    var = jnp.mean((x - mu) ** 2, -1, keepdims=True)
    hh = (x - mu) / jnp.sqrt(var + EPS) * g_ref[...] + b_ref[...]
    a = jnp.dot(hh, w1_ref[...], preferred_element_type=jnp.float32) + b1_ref[...]
    ge = 0.5 * a * (1.0 + jax.lax.erf(a * (2.0 ** -0.5)))
    part = jnp.dot(ge, w2_ref[...], preferred_element_type=jnp.float32)

    @pl.when(kb == 0)
    def _():
        base = x2_ref[...] + b2_ref[...]
        if add_y1:
            base = base + x
        out_ref[...] = base + part

    @pl.when(kb != 0)
    def _():
        out_ref[...] += part


def _ffn(y1, x2, g, b, W1, b1, W2, b2, S, add_y1, blk=256, kb=1024):
    DF = W1.shape[-1]
    body = functools.partial(_ffn_body, add_y1=add_y1)
    return pl.pallas_call(
        body,
        grid=(S // blk, DF // kb),
        in_specs=[
            pl.BlockSpec((blk, D), lambda i, j: (i, 0)),
            pl.BlockSpec((blk, D), lambda i, j: (i, 0)),
            pl.BlockSpec((1, D), lambda i, j: (0, 0)),
            pl.BlockSpec((1, D), lambda i, j: (0, 0)),
            pl.BlockSpec((D, kb), lambda i, j: (0, j)),
            pl.BlockSpec((1, kb), lambda i, j: (0, j)),
            pl.BlockSpec((kb, D), lambda i, j: (j, 0)),
            pl.BlockSpec((1, D), lambda i, j: (0, 0)),
        ],
        out_specs=pl.BlockSpec((blk, D), lambda i, j: (i, 0)),
        out_shape=jax.ShapeDtypeStruct((S, D), jnp.float32),
        compiler_params=pltpu.CompilerParams(
            dimension_semantics=("arbitrary", "arbitrary")),
    )(y1, x2, g.reshape(1, D), b.reshape(1, D), W1, b1.reshape(1, DF), W2,
      b2.reshape(1, D))


_NW = 32
_CHUNK = 512


def _sc_gather(table, idx, W, dtype=jnp.float32):
    N = idx.shape[0]
    n_per = N // _NW
    nch = n_per // _CHUNK
    mesh = plsc.VectorSubcoreMesh(core_axis_name="c", subcore_axis_name="s")

    @functools.partial(
        pl.kernel, mesh=mesh,
        out_type=jax.ShapeDtypeStruct((N, W), dtype),
        scratch_types=[
            pltpu.VMEM((_CHUNK,), jnp.int32),
            pltpu.VMEM((_CHUNK, W), dtype),
            pltpu.SemaphoreType.DMA,
        ],
    )
    def k(table_hbm, idx_hbm, out_hbm, idx_v, rows_v, sem):
        wid = lax.axis_index("s") * 2 + lax.axis_index("c")
        base = wid * n_per
        for c in range(nch):
            off = base + c * _CHUNK
            pltpu.sync_copy(idx_hbm.at[pl.ds(off, _CHUNK)], idx_v)
            pltpu.async_copy(table_hbm.at[idx_v], rows_v, sem).wait()
            pltpu.sync_copy(rows_v, out_hbm.at[pl.ds(off, _CHUNK)])

    return k(table, idx)


def _sc_scatter(rows, idx, W, dtype=jnp.float32):
    N = idx.shape[0]
    n_per = N // _NW
    nch = n_per // _CHUNK
    mesh = plsc.VectorSubcoreMesh(core_axis_name="c", subcore_axis_name="s")

    @functools.partial(
        pl.kernel, mesh=mesh,
        out_type=jax.ShapeDtypeStruct((N, W), dtype),
        scratch_types=[
            pltpu.VMEM((_CHUNK,), jnp.int32),
            pltpu.VMEM((_CHUNK, W), dtype),
            pltpu.SemaphoreType.DMA,
        ],
    )
    def k(rows_hbm, idx_hbm, out_hbm, idx_v, rows_v, sem):
        wid = lax.axis_index("s") * 2 + lax.axis_index("c")
        base = wid * n_per
        for c in range(nch):
            off = base + c * _CHUNK
            pltpu.sync_copy(idx_hbm.at[pl.ds(off, _CHUNK)], idx_v)
            pltpu.sync_copy(rows_hbm.at[pl.ds(off, _CHUNK)], rows_v)
            pltpu.async_copy(rows_v, out_hbm.at[idx_v], sem).wait()

    return k(rows, idx)


def _gather_rows(qkv, st, S):
    gidx = (st + (jnp.arange(H, dtype=jnp.int32) * S)[:, None]).reshape(-1)
    out = _sc_gather(qkv.reshape(H * S, 2 * DH), gidx, 2 * DH)
    return out.reshape(H, st.shape[1], 2 * DH)


def _attn_layer(x1, x2, g, b, Wqk, Wv, Wo, bo, rot, S):
    nc = NHASH * NBKT
    n = NHASH * S
    qk2d, v2d = _qkv_proj(x2, g, b, Wqk, Wv, S)
    qkh = qk2d.reshape(S, H, DH).transpose(1, 0, 2)
    vh = v2d.reshape(S, H, DH).transpose(1, 0, 2)
    rotf = jnp.concatenate([rot, -rot], axis=-1).reshape(DH, NHASH * 2 * HB)
    bid = _bucket_ids(qkh, rotf, S)
    b_flat = bid.transpose(0, 2, 1).reshape(H, n)
    pos = (jnp.arange(n, dtype=jnp.int32) % S)
    keys = b_flat * S + pos[None, :]
    sticker = jnp.argsort(keys, axis=-1).astype(jnp.int32)
    st = (sticker % S).astype(jnp.int32)
    didx = (sticker
            + (jnp.arange(H, dtype=jnp.int32) * n)[:, None]).reshape(-1)
    qkv = jnp.concatenate([qkh, vh], axis=-1)
    sqkv = _gather_rows(qkv, st, S).reshape(H, nc, BS, 2 * DH)
    stc = st.reshape(H, nc, BS)
    attn = _attention(sqkv, stc, nc)
    u = _sc_scatter(attn.reshape(H * n, _OW), didx, _OW)
    u = u.reshape(H, NHASH, S, _OW)
    return _combine_wo(u, Wo, x1, bo, S)


def kernel(x, gf, bf, Wqk, Wv, Wo, bo, gg, bg, W1, b1, W2, b2, rot):
    S = x.shape[1]
    x1 = x[0]
    x2 = x[0]
    depth = gf.shape[0]
    for l in range(depth):
        y1 = _attn_layer(x1, x2, gf[l], bf[l], Wqk[l], Wv[l], Wo[l], bo[l],
                         rot[l], S)
        y2 = _ffn(y1, x2, gg[l], bg[l], W1[l], b1[l], W2[l], b2[l], S,
                  add_y1=(l == depth - 1))
        x1, x2 = y1, y2
    return x2[None]


# speedup vs baseline: 2.0255x; 1.0479x over previous
import functools

import jax
import jax.numpy as jnp
from jax import lax
from jax.experimental import pallas as pl
from jax.experimental.pallas import tpu as pltpu
from jax.experimental.pallas import tpu_sc as plsc

D = 1024
H = 16
DH = 64
NHASH = 4
NBKT = 32
HB = 16
BS = 64
EPS = 1e-5
_OW = DH + 64


def _qkv_body(x_ref, g_ref, b_ref, wqk_ref, wv_ref, qk_ref, v_ref):
    x = x_ref[...]
    mu = jnp.mean(x, -1, keepdims=True)
    var = jnp.mean((x - mu) ** 2, -1, keepdims=True)
    h = (x - mu) / jnp.sqrt(var + EPS) * g_ref[...] + b_ref[...]
    qk_ref[...] = jnp.dot(h, wqk_ref[...], preferred_element_type=jnp.float32)
    v_ref[...] = jnp.dot(h, wv_ref[...], preferred_element_type=jnp.float32)


def _qkv_proj(x2, g, b, Wqk, Wv, S, blk=256):
    grid = (S // blk,)
    return pl.pallas_call(
        _qkv_body,
        grid=grid,
        in_specs=[
            pl.BlockSpec((blk, D), lambda i: (i, 0)),
            pl.BlockSpec((1, D), lambda i: (0, 0)),
            pl.BlockSpec((1, D), lambda i: (0, 0)),
            pl.BlockSpec((D, D), lambda i: (0, 0)),
            pl.BlockSpec((D, D), lambda i: (0, 0)),
        ],
        out_specs=[
            pl.BlockSpec((blk, D), lambda i: (i, 0)),
            pl.BlockSpec((blk, D), lambda i: (i, 0)),
        ],
        out_shape=[
            jax.ShapeDtypeStruct((S, D), jnp.float32),
            jax.ShapeDtypeStruct((S, D), jnp.float32),
        ],
    )(x2, g.reshape(1, D), b.reshape(1, D), Wqk, Wv)


def _bucket_body(qk_ref, rot_ref, bid_ref):
    S = qk_ref.shape[1]
    r = jnp.dot(qk_ref[0], rot_ref[...], preferred_element_type=jnp.float32)
    cols = []
    for h in range(NHASH):
        seg = r[:, h * 2 * HB:(h + 1) * 2 * HB]
        b = jnp.argmax(seg, axis=-1, keepdims=True).astype(jnp.int32)
        cols.append(b + h * NBKT)
    bid_ref[0] = jnp.concatenate(cols, axis=-1)


def _bucket_ids(qk_heads, rotf, S):
    return pl.pallas_call(
        _bucket_body,
        grid=(H,),
        in_specs=[
            pl.BlockSpec((1, S, DH), lambda h: (h, 0, 0)),
            pl.BlockSpec((DH, NHASH * 2 * HB), lambda h: (0, 0)),
        ],
        out_specs=pl.BlockSpec((1, S, NHASH), lambda h: (h, 0, 0)),
        out_shape=jax.ShapeDtypeStruct((H, S, NHASH), jnp.int32),
    )(qk_heads, rotf)


_CSR = 256


def _count_body(b_ref, dest_ref, *, n, nk):
    g = pl.program_id(0)
    b_col = b_ref[0]
    iota_k = jax.lax.broadcasted_iota(jnp.int32, (1, nk), 1)
    O = (b_col == iota_k).astype(jnp.float32)
    nb = n // _CSR
    O3 = O.reshape(nb, _CSR, nk)
    Hg = jnp.sum(O3, axis=1)
    gi = jax.lax.broadcasted_iota(jnp.int32, (nb, nb), 0)
    gj = jax.lax.broadcasted_iota(jnp.int32, (nb, nb), 1)
    Tg = (gj < gi).astype(jnp.float32)
    Hcum = jnp.dot(Tg, Hg, preferred_element_type=jnp.float32)
    total = jnp.sum(Hg, axis=0, keepdims=True)
    ki = jax.lax.broadcasted_iota(jnp.int32, (nk, nk), 0)
    kj = jax.lax.broadcasted_iota(jnp.int32, (nk, nk), 1)
    Mu = (ki < kj).astype(jnp.float32)
    startk = jnp.dot(total, Mu, preferred_element_type=jnp.float32)
    offs = Hcum + startk
    ri = jax.lax.broadcasted_iota(jnp.int32, (_CSR, _CSR), 0)
    rj = jax.lax.broadcasted_iota(jnp.int32, (_CSR, _CSR), 1)
    L = (rj < ri).astype(jnp.float32)
    for blk in range(nb):
        Og = O3[blk]
        P = jnp.dot(L, Og, preferred_element_type=jnp.float32)
        dg = jnp.sum(Og * (P + offs[blk:blk + 1, :]), axis=1, keepdims=True)
        dest_ref[0, pl.ds(blk * _CSR, _CSR)] = dg.astype(jnp.int32)


def _count_sort(b_flat):
    n = b_flat.shape[1]
    nk = NHASH * NBKT
    body = functools.partial(_count_body, n=n, nk=nk)
    out = pl.pallas_call(
        body,
        grid=(H,),
        in_specs=[pl.BlockSpec((1, n, 1), lambda h: (h, 0, 0))],
        out_specs=pl.BlockSpec((1, n, 1), lambda h: (h, 0, 0)),
        out_shape=jax.ShapeDtypeStruct((H, n, 1), jnp.int32),
    )(b_flat.reshape(H, n, 1))
    return out.reshape(H, n)


def _attn_body(sqkv_ref, stc_ref, out_ref, *, nc, cb):
    ncb = nc // cb

    def prep(blk):
        qk = blk[..., :DH]
        v = blk[..., DH:]
        nrm = jnp.sqrt(jnp.sum(qk * qk, -1, keepdims=True)) + 1e-9
        return qk / nrm, v

    def body(i, _):
        c0 = i * cb
        blk = sqkv_ref[0, pl.ds(c0, cb)]
        q = blk[..., :DH]
        prev_c = jnp.where(c0 == 0, nc - 1, c0 - 1)
        pblk = sqkv_ref[0, pl.ds(prev_c, 1)]
        k_c, v_c = prep(blk)
        k_p1, v_p1 = prep(pblk)
        k_prev = jnp.concatenate([k_p1, k_c[:-1]], 0)
        v_prev = jnp.concatenate([v_p1, v_c[:-1]], 0)
        t_c = stc_ref[0, pl.ds(c0, cb)]
        t_p1 = stc_ref[0, pl.ds(prev_c, 1)]
        t_prev = jnp.concatenate([t_p1, t_c[:-1]], 0)
        bk = jnp.concatenate([k_c, k_prev], 1)
        bv = jnp.concatenate([v_c, v_prev], 1)
        bt = jnp.concatenate([t_c, t_prev], 1)
        dots = jax.lax.dot_general(
            q, bk, (((2,), (2,)), ((0,), (0,))),
            preferred_element_type=jnp.float32) * (DH ** -0.5)
        mask = t_c[:, :, None] == bt[:, None, :]
        dots = jnp.where(mask, -1e5, dots)
        m = jnp.max(dots, -1, keepdims=True)
        lse = m + jnp.log(jnp.sum(jnp.exp(dots - m), -1, keepdims=True))
        p = jnp.exp(dots - lse)
        o = jax.lax.dot_general(
            p, bv, (((2,), (1,)), ((0,), (0,))),
            preferred_element_type=jnp.float32)
        out_ref[0, pl.ds(c0, cb), :, :DH] = o
        out_ref[0, pl.ds(c0, cb), :, DH:DH + 1] = lse
        return 0

    jax.lax.fori_loop(0, ncb, body, 0)


def _attention(sqkv, stc, nc, cb=16):
    nh = sqkv.shape[0]
    body = functools.partial(_attn_body, nc=nc, cb=cb)
    return pl.pallas_call(
        body,
        grid=(nh,),
        in_specs=[
            pl.BlockSpec((1, nc, BS, 2 * DH), lambda h: (h, 0, 0, 0)),
            pl.BlockSpec((1, nc, BS), lambda h: (h, 0, 0)),
        ],
        out_specs=pl.BlockSpec((1, nc, BS, _OW), lambda h: (h, 0, 0, 0)),
        out_shape=jax.ShapeDtypeStruct((nh, nc, BS, _OW), jnp.float32),
    )(sqkv, stc)


def _comb_body(u_ref, wo_ref, base_ref, out_ref):
    h = pl.program_id(0)
    u = u_ref[0]
    so = u[..., :DH]
    sl = u[..., DH:DH + 1]
    m = jnp.max(sl, 0, keepdims=True)
    lse = m + jnp.log(jnp.sum(jnp.exp(sl - m), 0, keepdims=True))
    p = jnp.exp(sl - lse)
    o = jnp.sum(so * p, 0)
    acc = jnp.dot(o, wo_ref[0], preferred_element_type=jnp.float32)

    @pl.when(h == 0)
    def _():
        out_ref[...] = base_ref[...] + acc

    @pl.when(h != 0)
    def _():
        out_ref[...] += acc


def _combine_wo(u, wo3, base, S):
    nh = u.shape[0]
    return pl.pallas_call(
        _comb_body,
        grid=(nh,),
        in_specs=[
            pl.BlockSpec((1, NHASH, S, _OW), lambda h: (h, 0, 0, 0)),
            pl.BlockSpec((1, DH, D), lambda h: (h, 0, 0)),
            pl.BlockSpec((S, D), lambda h: (0, 0)),
        ],
        out_specs=pl.BlockSpec((S, D), lambda h: (0, 0)),
        out_shape=jax.ShapeDtypeStruct((S, D), jnp.float32),
        compiler_params=pltpu.CompilerParams(
            dimension_semantics=("arbitrary",)),
    )(u, wo3, base)


def _ffn_body(y1_ref, x2_ref, g_ref, b_ref, w1_ref, b1_ref, w2_ref, b2_ref,
              out_ref, *, add_y1):
    kb = pl.program_id(1)
    x = y1_ref[...]
    mu = jnp.mean(x, -1, keepdims=True)
    var = jnp.mean((x - mu) ** 2, -1, keepdims=True)
    hh = (x - mu) / jnp.sqrt(var + EPS) * g_ref[...] + b_ref[...]
    a = jnp.dot(hh, w1_ref[...], preferred_element_type=jnp.float32) + b1_ref[...]
    ge = 0.5 * a * (1.0 + jax.lax.erf(a * (2.0 ** -0.5)))
    part = jnp.dot(ge, w2_ref[...], preferred_element_type=jnp.float32)

    @pl.when(kb == 0)
    def _():
        base = x2_ref[...] + b2_ref[...]
        if add_y1:
            base = base + x
        out_ref[...] = base + part

    @pl.when(kb != 0)
    def _():
        out_ref[...] += part


def _ffn(y1, x2, g, b, W1, b1, W2, b2, S, add_y1, blk=256, kb=1024):
    DF = W1.shape[-1]
    body = functools.partial(_ffn_body, add_y1=add_y1)
    return pl.pallas_call(
        body,
        grid=(S // blk, DF // kb),
        in_specs=[
            pl.BlockSpec((blk, D), lambda i, j: (i, 0)),
            pl.BlockSpec((blk, D), lambda i, j: (i, 0)),
            pl.BlockSpec((1, D), lambda i, j: (0, 0)),
            pl.BlockSpec((1, D), lambda i, j: (0, 0)),
            pl.BlockSpec((D, kb), lambda i, j: (0, j)),
            pl.BlockSpec((1, kb), lambda i, j: (0, j)),
            pl.BlockSpec((kb, D), lambda i, j: (j, 0)),
            pl.BlockSpec((1, D), lambda i, j: (0, 0)),
        ],
        out_specs=pl.BlockSpec((blk, D), lambda i, j: (i, 0)),
        out_shape=jax.ShapeDtypeStruct((S, D), jnp.float32),
        compiler_params=pltpu.CompilerParams(
            dimension_semantics=("arbitrary", "arbitrary")),
    )(y1, x2, g.reshape(1, D), b.reshape(1, D), W1, b1.reshape(1, DF), W2,
      b2.reshape(1, D))


_NW = 32
_CHUNK = 512


def _sc_gather(table, idx, W, dtype=jnp.float32):
    N = idx.shape[0]
    n_per = N // _NW
    nch = n_per // _CHUNK
    mesh = plsc.VectorSubcoreMesh(core_axis_name="c", subcore_axis_name="s")

    @functools.partial(
        pl.kernel, mesh=mesh,
        out_type=jax.ShapeDtypeStruct((N, W), dtype),
        scratch_types=[
            pltpu.VMEM((_CHUNK,), jnp.int32),
            pltpu.VMEM((_CHUNK, W), dtype),
            pltpu.SemaphoreType.DMA,
        ],
    )
    def k(table_hbm, idx_hbm, out_hbm, idx_v, rows_v, sem):
        wid = lax.axis_index("s") * 2 + lax.axis_index("c")
        base = wid * n_per
        for c in range(nch):
            off = base + c * _CHUNK
            pltpu.sync_copy(idx_hbm.at[pl.ds(off, _CHUNK)], idx_v)
            pltpu.async_copy(table_hbm.at[idx_v], rows_v, sem).wait()
            pltpu.sync_copy(rows_v, out_hbm.at[pl.ds(off, _CHUNK)])

    return k(table, idx)


def _sc_scatter(rows, idx, W, dtype=jnp.float32):
    N = idx.shape[0]
    n_per = N // _NW
    nch = n_per // _CHUNK
    mesh = plsc.VectorSubcoreMesh(core_axis_name="c", subcore_axis_name="s")

    @functools.partial(
        pl.kernel, mesh=mesh,
        out_type=jax.ShapeDtypeStruct((N, W), dtype),
        scratch_types=[
            pltpu.VMEM((_CHUNK,), jnp.int32),
            pltpu.VMEM((_CHUNK, W), dtype),
            pltpu.SemaphoreType.DMA,
        ],
    )
    def k(rows_hbm, idx_hbm, out_hbm, idx_v, rows_v, sem):
        wid = lax.axis_index("s") * 2 + lax.axis_index("c")
        base = wid * n_per
        for c in range(nch):
            off = base + c * _CHUNK
            pltpu.sync_copy(idx_hbm.at[pl.ds(off, _CHUNK)], idx_v)
            pltpu.sync_copy(rows_hbm.at[pl.ds(off, _CHUNK)], rows_v)
            pltpu.async_copy(rows_v, out_hbm.at[idx_v], sem).wait()

    return k(rows, idx)


def _gather_rows(qkv, st, S):
    nh = qkv.shape[0]
    gidx = (st + (jnp.arange(nh, dtype=jnp.int32) * S)[:, None]).reshape(-1)
    out = _sc_gather(qkv.reshape(nh * S, 2 * DH), gidx, 2 * DH)
    return out.reshape(nh, st.shape[1], 2 * DH)


def _attn_layer(x1, x2, g, b, Wqk, Wv, Wo, bo, rot, S):
    nc = NHASH * NBKT
    n = NHASH * S
    qk2d, v2d = _qkv_proj(x2, g, b, Wqk, Wv, S)
    qkh = qk2d.reshape(S, H, DH).transpose(1, 0, 2)
    vh = v2d.reshape(S, H, DH).transpose(1, 0, 2)
    rotf = jnp.concatenate([rot, -rot], axis=-1).reshape(DH, NHASH * 2 * HB)
    bid = _bucket_ids(qkh, rotf, S)
    b_flat = bid.transpose(0, 2, 1).reshape(H, n)
    pos = (jnp.arange(n, dtype=jnp.int32) % S)
    keys = b_flat * S + pos[None, :]
    sticker = jnp.argsort(keys, axis=-1).astype(jnp.int32)
    st = (sticker % S).astype(jnp.int32)
    qkv = jnp.concatenate([qkh, vh], axis=-1)
    wo3 = Wo.reshape(H, DH, D)
    ng = 2
    gh = H // ng
    us = []
    for g in range(ng):
        hs = slice(g * gh, (g + 1) * gh)
        sqkv = _gather_rows(qkv[hs], st[hs], S).reshape(gh, nc, BS, 2 * DH)
        attn = _attention(sqkv, st[hs].reshape(gh, nc, BS), nc)
        didx = (sticker[hs]
                + (jnp.arange(gh, dtype=jnp.int32) * n)[:, None]).reshape(-1)
        u = _sc_scatter(attn.reshape(gh * n, _OW), didx, _OW)
        us.append(u.reshape(gh, NHASH, S, _OW))
    base = x1 + bo[None, :]
    for g in range(ng):
        hs = slice(g * gh, (g + 1) * gh)
        base = _combine_wo(us[g], wo3[hs], base, S)
    return base


def kernel(x, gf, bf, Wqk, Wv, Wo, bo, gg, bg, W1, b1, W2, b2, rot):
    S = x.shape[1]
    x1 = x[0]
    x2 = x[0]
    depth = gf.shape[0]
    for l in range(depth):
        y1 = _attn_layer(x1, x2, gf[l], bf[l], Wqk[l], Wv[l], Wo[l], bo[l],
                         rot[l], S)
        y2 = _ffn(y1, x2, gg[l], bg[l], W1[l], b1[l], W2[l], b2[l], S,
                  add_y1=(l == depth - 1))
        x1, x2 = y1, y2
    return x2[None]
